# baseline (speedup 1.0000x reference)
import functools

import jax
import jax.numpy as jnp
from jax import lax
from jax.experimental import pallas as pl
from jax.experimental.pallas import tpu as pltpu
from jax.experimental.pallas import tpu_sc as plsc

NUM_WORKERS = 32
LANES = 16
CHUNK = 16384


def _spline_rows(widths, heights, slopes):
    nb = widths.shape[0]
    edges = jnp.cumsum(widths)
    x_low = jnp.concatenate([jnp.zeros((1,), widths.dtype), edges[:-1]])
    inv_w = 1.0 / (widths + 1e-8)
    y_low = heights[:-1]
    dy = heights[1:] - heights[:-1]
    s_lo = slopes[:-1]
    s_sum = slopes[:-1] + slopes[1:]
    pad = jnp.zeros((LANES - nb,), jnp.float32)

    def row(v):
        return jnp.concatenate([v.astype(jnp.float32), pad])

    rows = [
        row(x_low),
        row(inv_w),
        row(y_low),
        row(dy),
        row(s_lo - 2.0),
        row(s_sum),
        row(2.0 * (s_sum - 2.0)),
    ]
    for i in range(4):
        rows.append(jnp.full((LANES,), edges[i], jnp.float32))
    rows.append(jnp.zeros((LANES,), jnp.float32))
    return jnp.concatenate(rows)


ROWS = 32


@functools.lru_cache(maxsize=None)
def _make_sc_kernel(m_sc, m):
    rows_w = m_sc // NUM_WORKERS
    assert rows_w * NUM_WORKERS == m_sc and rows_w % (2 * ROWS) == 0
    n_chunks = rows_w // ROWS
    mesh = plsc.VectorSubcoreMesh(core_axis_name="c", subcore_axis_name="s")

    @functools.partial(
        pl.kernel,
        out_type=jax.ShapeDtypeStruct((m_sc, 512), jnp.float32),
        mesh=mesh,
        compiler_params=pltpu.CompilerParams(use_tc_tiling_on_sc=True),
        scratch_types=[
            pltpu.VMEM((3 * LANES,), jnp.float32),
            pltpu.VMEM((ROWS, 512), jnp.float32),
            pltpu.VMEM((ROWS, 512), jnp.float32),
            pltpu.VMEM((ROWS, 512), jnp.float32),
            pltpu.VMEM((ROWS, 512), jnp.float32),
            pltpu.SemaphoreType.DMA,
            pltpu.SemaphoreType.DMA,
            pltpu.SemaphoreType.DMA,
            pltpu.SemaphoreType.DMA,
        ],
    )
    def tone(x_hbm, w_hbm, h_hbm, s_hbm, out_hbm, whs_v, in0, in1, out0, out1,
             si0, si1, so0, so1):
        wid = lax.axis_index("s") * 2 + lax.axis_index("c")
        base = wid * rows_w
        pltpu.sync_copy(w_hbm, whs_v.at[pl.ds(0, 5)])
        pltpu.sync_copy(h_hbm, whs_v.at[pl.ds(LANES, 6)])
        pltpu.sync_copy(s_hbm, whs_v.at[pl.ds(2 * LANES, 6)])

        gdims = lax.GatherDimensionNumbers(
            offset_dims=(), collapsed_slice_dims=(0,), start_index_map=(0,))

        def _gather(tab, idx):
            return lax.gather(
                tab, idx[:, None], gdims, (1,),
                mode=lax.GatherScatterMode.PROMISE_IN_BOUNDS)

        w_v = whs_v[pl.ds(0, LANES)]
        h_v = whs_v[pl.ds(LANES, LANES)]
        s_v = whs_v[pl.ds(2 * LANES, LANES)]
        lane = lax.iota(jnp.int32, LANES)
        lane1 = jnp.minimum(lane + 1, LANES - 1)
        edges = w_v
        for k in (1, 2, 4):
            shifted = _gather(edges, jnp.maximum(lane - k, 0))
            edges = edges + jnp.where(lane >= k, shifted, 0.0)
        t_xlow = jnp.where(lane == 0, 0.0,
                           _gather(edges, jnp.maximum(lane - 1, 0)))
        t_invw = 1.0 / (w_v + 1e-8)
        t_ylow = h_v
        t_dy = _gather(h_v, lane1) - h_v
        s_hi = _gather(s_v, lane1)
        t_slm2 = s_v - 2.0
        t_ssum = s_v + s_hi
        t_tsm2 = t_ssum + t_ssum - 4.0
        e0 = _gather(edges, jnp.full((LANES,), 0, jnp.int32))
        e1 = _gather(edges, jnp.full((LANES,), 1, jnp.int32))
        e2 = _gather(edges, jnp.full((LANES,), 2, jnp.int32))
        e3 = _gather(edges, jnp.full((LANES,), 3, jnp.int32))
        two = jnp.full((LANES,), 2.0, jnp.float32)

        def run_chunk(in_v, out_v):
            def compute(i):
                r = i >> 5
                c = (i & 31) * LANES
                xv = in_v[r, pl.ds(c, LANES)]
                xf = jnp.minimum(jnp.maximum(xv, 0.0), 1.0)
                i0 = jnp.int32(0)
                idx = jnp.where(
                    xf > e3, jnp.int32(4),
                    jnp.where(xf > e2, jnp.int32(3),
                              jnp.where(xf > e1, jnp.int32(2),
                                        jnp.where(xf > e0, jnp.int32(1), i0))),
                )
                x_low = _gather(t_xlow, idx)
                inv_w = _gather(t_invw, idx)
                y_low = _gather(t_ylow, idx)
                dy = _gather(t_dy, idx)
                slm2 = _gather(t_slm2, idx)
                ssum = _gather(t_ssum, idx)
                tsm2 = _gather(t_tsm2, idx)
                t = (xf - x_low) * inv_w
                t2 = t * t
                num = (slm2 * t + two) * t
                den = ssum * t2 + (tsm2 * t + two)
                y = y_low + dy * (num / den)
                out_v[r, pl.ds(c, LANES)] = y

            plsc.parallel_loop(0, ROWS * 32, 1, unroll=4)(compute)

        n_pair = n_chunks // 2
        pltpu.async_copy(x_hbm.at[pl.ds(base, ROWS), :], in0, si0)
        pltpu.async_copy(x_hbm.at[pl.ds(base + ROWS, ROWS), :], in1, si1)

        def chunk_pair(k, _):
            off0 = base + k * (2 * ROWS)
            for (off, in_v, out_v, si, so) in (
                (off0, in0, out0, si0, so0),
                (off0 + ROWS, in1, out1, si1, so1),
            ):
                pltpu.make_async_copy(x_hbm.at[pl.ds(off, ROWS), :], in_v, si).wait()

                @pl.when(k > 0)
                def _():
                    pltpu.make_async_copy(
                        out_v, out_hbm.at[pl.ds(off, ROWS), :], so).wait()

                run_chunk(in_v, out_v)
                pltpu.async_copy(out_v, out_hbm.at[pl.ds(off, ROWS), :], so)

                @pl.when(k < n_pair - 1)
                def _():
                    pltpu.async_copy(
                        x_hbm.at[pl.ds(off + 2 * ROWS, ROWS), :], in_v, si)
            return 0

        lax.fori_loop(0, n_pair, chunk_pair, 0)
        last0 = base + (n_chunks - 2) * ROWS
        pltpu.make_async_copy(out0, out_hbm.at[pl.ds(last0, ROWS), :], so0).wait()
        pltpu.make_async_copy(
            out1, out_hbm.at[pl.ds(last0 + ROWS, ROWS), :], so1).wait()

    return tone


TC_BLOCK_ROWS = 512


@functools.lru_cache(maxsize=None)
def _make_tc_kernel(m, m_sc):
    n_blocks = (m - m_sc) // TC_BLOCK_ROWS
    assert n_blocks * TC_BLOCK_ROWS == m - m_sc
    blk0 = m_sc // TC_BLOCK_ROWS

    def body(w_ref, h_ref, s_ref, x_ref, o_ref):
        edges = [w_ref[0]]
        for b in range(1, 5):
            edges.append(edges[-1] + w_ref[b])
        x_low = [0.0] + edges[:4]
        inv_w = [1.0 / (w_ref[b] + 1e-8) for b in range(5)]
        y_low = [h_ref[b] for b in range(5)]
        dy = [h_ref[b + 1] - h_ref[b] for b in range(5)]
        slm2 = [s_ref[b] - 2.0 for b in range(5)]
        ssum = [s_ref[b] + s_ref[b + 1] for b in range(5)]
        tsm2 = [2.0 * (v - 2.0) for v in ssum]

        xv = x_ref[...]
        xf = jnp.minimum(jnp.maximum(xv, 0.0), 1.0)
        m0 = xf > edges[0]
        m1 = xf > edges[1]
        m2 = xf > edges[2]
        m3 = xf > edges[3]

        def sel(p):
            return jnp.where(
                m3, p[4],
                jnp.where(m2, p[3], jnp.where(m1, p[2], jnp.where(m0, p[1], p[0]))))

        xl = sel(x_low)
        iw = sel(inv_w)
        yl = sel(y_low)
        dyv = sel(dy)
        a = sel(slm2)
        sm = sel(ssum)
        ts = sel(tsm2)
        t = (xf - xl) * iw
        t2 = t * t
        num = (a * t + 2.0) * t
        den = sm * t2 + (ts * t + 2.0)
        o_ref[...] = yl + dyv * (num / den)

    return pl.pallas_call(
        body,
        grid=(n_blocks,),
        in_specs=[
            pl.BlockSpec(memory_space=pltpu.SMEM),
            pl.BlockSpec(memory_space=pltpu.SMEM),
            pl.BlockSpec(memory_space=pltpu.SMEM),
            pl.BlockSpec((TC_BLOCK_ROWS, 512), lambda i: (blk0 + i, 0)),
        ],
        out_specs=pl.BlockSpec((TC_BLOCK_ROWS, 512), lambda i: (blk0 + i, 0)),
        out_shape=jax.ShapeDtypeStruct((m, 512), jnp.float32),
    )


SC_ROWS_TOTAL = 8192


def kernel(x, widths, heights, slopes):
    orig_shape = x.shape
    x2 = x.reshape(-1, 512)
    m = x2.shape[0]
    m_sc = SC_ROWS_TOTAL if m % (2 * 32 * 32) == 0 and m > SC_ROWS_TOTAL else m
    if m_sc < m:
        sc_out = _make_sc_kernel(m_sc, m)(x2, widths, heights, slopes)
        tc_out = _make_tc_kernel(m, m_sc)(widths, heights, slopes, x2)
        out = jax.lax.dynamic_update_slice(tc_out, sc_out, (0, 0))
    else:
        out = _make_sc_kernel(m, m)(x2, widths, heights, slopes)
    return out.reshape(orig_shape)


# speedup vs baseline: 1.6587x; 1.6587x over previous
import functools

import jax
import jax.numpy as jnp
from jax import lax
from jax.experimental import pallas as pl
from jax.experimental.pallas import tpu as pltpu
from jax.experimental.pallas import tpu_sc as plsc

NUM_WORKERS = 32
LANES = 16
CHUNK = 16384


def _spline_rows(widths, heights, slopes):
    nb = widths.shape[0]
    edges = jnp.cumsum(widths)
    x_low = jnp.concatenate([jnp.zeros((1,), widths.dtype), edges[:-1]])
    inv_w = 1.0 / (widths + 1e-8)
    y_low = heights[:-1]
    dy = heights[1:] - heights[:-1]
    s_lo = slopes[:-1]
    s_sum = slopes[:-1] + slopes[1:]
    pad = jnp.zeros((LANES - nb,), jnp.float32)

    def row(v):
        return jnp.concatenate([v.astype(jnp.float32), pad])

    rows = [
        row(x_low),
        row(inv_w),
        row(y_low),
        row(dy),
        row(s_lo - 2.0),
        row(s_sum),
        row(2.0 * (s_sum - 2.0)),
    ]
    for i in range(4):
        rows.append(jnp.full((LANES,), edges[i], jnp.float32))
    rows.append(jnp.zeros((LANES,), jnp.float32))
    return jnp.concatenate(rows)


ROWS = 32


@functools.lru_cache(maxsize=None)
def _make_sc_kernel(m_sc, m):
    rows_w = m_sc // NUM_WORKERS
    assert rows_w * NUM_WORKERS == m_sc and rows_w % (2 * ROWS) == 0
    n_chunks = rows_w // ROWS
    mesh = plsc.VectorSubcoreMesh(core_axis_name="c", subcore_axis_name="s")

    @functools.partial(
        pl.kernel,
        out_type=jax.ShapeDtypeStruct((m_sc, 512), jnp.float32),
        mesh=mesh,
        compiler_params=pltpu.CompilerParams(use_tc_tiling_on_sc=True),
        scratch_types=[
            pltpu.VMEM((3 * LANES,), jnp.float32),
            pltpu.VMEM((ROWS, 512), jnp.float32),
            pltpu.VMEM((ROWS, 512), jnp.float32),
            pltpu.VMEM((ROWS, 512), jnp.float32),
            pltpu.VMEM((ROWS, 512), jnp.float32),
            pltpu.SemaphoreType.DMA,
            pltpu.SemaphoreType.DMA,
            pltpu.SemaphoreType.DMA,
            pltpu.SemaphoreType.DMA,
        ],
    )
    def tone(x_hbm, w_hbm, h_hbm, s_hbm, out_hbm, whs_v, in0, in1, out0, out1,
             si0, si1, so0, so1):
        wid = lax.axis_index("s") * 2 + lax.axis_index("c")
        base = wid * rows_w
        pltpu.sync_copy(w_hbm, whs_v.at[pl.ds(0, 5)])
        pltpu.sync_copy(h_hbm, whs_v.at[pl.ds(LANES, 6)])
        pltpu.sync_copy(s_hbm, whs_v.at[pl.ds(2 * LANES, 6)])

        gdims = lax.GatherDimensionNumbers(
            offset_dims=(), collapsed_slice_dims=(0,), start_index_map=(0,))

        def _gather(tab, idx):
            return lax.gather(
                tab, idx[:, None], gdims, (1,),
                mode=lax.GatherScatterMode.PROMISE_IN_BOUNDS)

        w_v = whs_v[pl.ds(0, LANES)]
        h_v = whs_v[pl.ds(LANES, LANES)]
        s_v = whs_v[pl.ds(2 * LANES, LANES)]
        lane = lax.iota(jnp.int32, LANES)
        lane1 = jnp.minimum(lane + 1, LANES - 1)
        edges = w_v
        for k in (1, 2, 4):
            shifted = _gather(edges, jnp.maximum(lane - k, 0))
            edges = edges + jnp.where(lane >= k, shifted, 0.0)
        t_xlow = jnp.where(lane == 0, 0.0,
                           _gather(edges, jnp.maximum(lane - 1, 0)))
        t_invw = 1.0 / (w_v + 1e-8)
        t_ylow = h_v
        t_dy = _gather(h_v, lane1) - h_v
        s_hi = _gather(s_v, lane1)
        t_slm2 = s_v - 2.0
        t_ssum = s_v + s_hi
        t_tsm2 = t_ssum + t_ssum - 4.0
        e0 = _gather(edges, jnp.full((LANES,), 0, jnp.int32))
        e1 = _gather(edges, jnp.full((LANES,), 1, jnp.int32))
        e2 = _gather(edges, jnp.full((LANES,), 2, jnp.int32))
        e3 = _gather(edges, jnp.full((LANES,), 3, jnp.int32))
        two = jnp.full((LANES,), 2.0, jnp.float32)

        def run_chunk(in_v, out_v):
            def compute(i):
                r = i >> 5
                c = (i & 31) * LANES
                xv = in_v[r, pl.ds(c, LANES)]
                xf = jnp.minimum(jnp.maximum(xv, 0.0), 1.0)
                i0 = jnp.int32(0)
                idx = jnp.where(
                    xf > e3, jnp.int32(4),
                    jnp.where(xf > e2, jnp.int32(3),
                              jnp.where(xf > e1, jnp.int32(2),
                                        jnp.where(xf > e0, jnp.int32(1), i0))),
                )
                x_low = _gather(t_xlow, idx)
                inv_w = _gather(t_invw, idx)
                y_low = _gather(t_ylow, idx)
                dy = _gather(t_dy, idx)
                slm2 = _gather(t_slm2, idx)
                ssum = _gather(t_ssum, idx)
                tsm2 = _gather(t_tsm2, idx)
                t = (xf - x_low) * inv_w
                t2 = t * t
                num = (slm2 * t + two) * t
                den = ssum * t2 + (tsm2 * t + two)
                y = y_low + dy * (num / den)
                out_v[r, pl.ds(c, LANES)] = y

            plsc.parallel_loop(0, ROWS * 32, 1, unroll=4)(compute)

        n_pair = n_chunks // 2
        pltpu.async_copy(x_hbm.at[pl.ds(base, ROWS), :], in0, si0)
        pltpu.async_copy(x_hbm.at[pl.ds(base + ROWS, ROWS), :], in1, si1)

        def chunk_pair(k, _):
            off0 = base + k * (2 * ROWS)
            for (off, in_v, out_v, si, so) in (
                (off0, in0, out0, si0, so0),
                (off0 + ROWS, in1, out1, si1, so1),
            ):
                pltpu.make_async_copy(x_hbm.at[pl.ds(off, ROWS), :], in_v, si).wait()

                @pl.when(k > 0)
                def _():
                    pltpu.make_async_copy(
                        out_v, out_hbm.at[pl.ds(off, ROWS), :], so).wait()

                run_chunk(in_v, out_v)
                pltpu.async_copy(out_v, out_hbm.at[pl.ds(off, ROWS), :], so)

                @pl.when(k < n_pair - 1)
                def _():
                    pltpu.async_copy(
                        x_hbm.at[pl.ds(off + 2 * ROWS, ROWS), :], in_v, si)
            return 0

        lax.fori_loop(0, n_pair, chunk_pair, 0)
        last0 = base + (n_chunks - 2) * ROWS
        pltpu.make_async_copy(out0, out_hbm.at[pl.ds(last0, ROWS), :], so0).wait()
        pltpu.make_async_copy(
            out1, out_hbm.at[pl.ds(last0 + ROWS, ROWS), :], so1).wait()

    return tone


TC_BLOCK_ROWS = 512


@functools.lru_cache(maxsize=None)
def _make_tc_kernel(m, m_sc):
    n_blocks = (m - m_sc) // TC_BLOCK_ROWS
    assert n_blocks * TC_BLOCK_ROWS == m - m_sc
    blk0 = m_sc // TC_BLOCK_ROWS

    def body(w_ref, h_ref, s_ref, x_ref, o_ref, p_ref):
        @pl.when(pl.program_id(0) == 0)
        def _():
            e = w_ref[0]
            p_ref[35] = e
            for b in range(1, 5):
                e = e + w_ref[b]
                p_ref[35 + b] = e
            for b in range(5):
                p_ref[0 + b] = p_ref[34 + b] if b > 0 else 0.0
                p_ref[5 + b] = 1.0 / (w_ref[b] + 1e-8)
                p_ref[10 + b] = h_ref[b]
                p_ref[15 + b] = h_ref[b + 1] - h_ref[b]
                p_ref[20 + b] = s_ref[b] - 2.0
                ss = s_ref[b] + s_ref[b + 1]
                p_ref[25 + b] = ss
                p_ref[30 + b] = 2.0 * (ss - 2.0)

        xv = x_ref[...]
        xf = jnp.minimum(jnp.maximum(xv, 0.0), 1.0)
        m0 = xf > p_ref[35]
        m1 = xf > p_ref[36]
        m2 = xf > p_ref[37]
        m3 = xf > p_ref[38]

        def sel(k):
            return jnp.where(
                m3, p_ref[k + 4],
                jnp.where(m2, p_ref[k + 3],
                          jnp.where(m1, p_ref[k + 2],
                                    jnp.where(m0, p_ref[k + 1], p_ref[k]))))

        xl = sel(0)
        iw = sel(5)
        yl = sel(10)
        dyv = sel(15)
        a = sel(20)
        sm = sel(25)
        ts = sel(30)
        t = (xf - xl) * iw
        t2 = t * t
        num = (a * t + 2.0) * t
        den = sm * t2 + (ts * t + 2.0)
        o_ref[...] = yl + dyv * (num / den)

    return pl.pallas_call(
        body,
        grid=(n_blocks,),
        in_specs=[
            pl.BlockSpec(memory_space=pltpu.SMEM),
            pl.BlockSpec(memory_space=pltpu.SMEM),
            pl.BlockSpec(memory_space=pltpu.SMEM),
            pl.BlockSpec((TC_BLOCK_ROWS, 512), lambda i: (blk0 + i, 0)),
        ],
        out_specs=pl.BlockSpec((TC_BLOCK_ROWS, 512), lambda i: (blk0 + i, 0)),
        out_shape=jax.ShapeDtypeStruct((m, 512), jnp.float32),
        scratch_shapes=[pltpu.SMEM((40,), jnp.float32)],
    )


SC_ROWS_TOTAL = 8192


def kernel(x, widths, heights, slopes):
    orig_shape = x.shape
    x2 = x.reshape(-1, 512)
    m = x2.shape[0]
    m_sc = SC_ROWS_TOTAL if m % (2 * 32 * 32) == 0 and m > SC_ROWS_TOTAL else m
    if m_sc < m:
        sc_out = _make_sc_kernel(m_sc, m)(x2, widths, heights, slopes)
        tc_out = _make_tc_kernel(m, m_sc)(widths, heights, slopes, x2)
        out = jax.lax.dynamic_update_slice(tc_out, sc_out, (0, 0))
    else:
        out = _make_sc_kernel(m, m)(x2, widths, heights, slopes)
    return out.reshape(orig_shape)


# speedup vs baseline: 1.6594x; 1.0004x over previous
import functools

import jax
import jax.numpy as jnp
from jax import lax
from jax.experimental import pallas as pl
from jax.experimental.pallas import tpu as pltpu
from jax.experimental.pallas import tpu_sc as plsc

NUM_WORKERS = 32
LANES = 16

ROWS = 32


@functools.lru_cache(maxsize=None)
def _make_sc_kernel(m_sc, m):
    rows_w = m_sc // NUM_WORKERS
    assert rows_w * NUM_WORKERS == m_sc and rows_w % (2 * ROWS) == 0
    n_chunks = rows_w // ROWS
    mesh = plsc.VectorSubcoreMesh(core_axis_name="c", subcore_axis_name="s")

    @functools.partial(
        pl.kernel,
        out_type=jax.ShapeDtypeStruct((m_sc, 512), jnp.float32),
        mesh=mesh,
        compiler_params=pltpu.CompilerParams(use_tc_tiling_on_sc=True),
        scratch_types=[
            pltpu.VMEM((3 * LANES,), jnp.float32),
            pltpu.VMEM((ROWS, 512), jnp.float32),
            pltpu.VMEM((ROWS, 512), jnp.float32),
            pltpu.VMEM((ROWS, 512), jnp.float32),
            pltpu.VMEM((ROWS, 512), jnp.float32),
            pltpu.SemaphoreType.DMA,
            pltpu.SemaphoreType.DMA,
            pltpu.SemaphoreType.DMA,
            pltpu.SemaphoreType.DMA,
        ],
    )
    def tone(x_hbm, w_hbm, h_hbm, s_hbm, out_hbm, whs_v, in0, in1, out0, out1,
             si0, si1, so0, so1):
        wid = lax.axis_index("s") * 2 + lax.axis_index("c")
        base = wid * rows_w
        pltpu.sync_copy(w_hbm, whs_v.at[pl.ds(0, 5)])
        pltpu.sync_copy(h_hbm, whs_v.at[pl.ds(LANES, 6)])
        pltpu.sync_copy(s_hbm, whs_v.at[pl.ds(2 * LANES, 6)])

        gdims = lax.GatherDimensionNumbers(
            offset_dims=(), collapsed_slice_dims=(0,), start_index_map=(0,))

        def _gather(tab, idx):
            return lax.gather(
                tab, idx[:, None], gdims, (1,),
                mode=lax.GatherScatterMode.PROMISE_IN_BOUNDS)

        w_v = whs_v[pl.ds(0, LANES)]
        h_v = whs_v[pl.ds(LANES, LANES)]
        s_v = whs_v[pl.ds(2 * LANES, LANES)]
        lane = lax.iota(jnp.int32, LANES)
        lane1 = jnp.minimum(lane + 1, LANES - 1)
        edges = w_v
        for k in (1, 2, 4):
            shifted = _gather(edges, jnp.maximum(lane - k, 0))
            edges = edges + jnp.where(lane >= k, shifted, 0.0)
        t_xlow = jnp.where(lane == 0, 0.0,
                           _gather(edges, jnp.maximum(lane - 1, 0)))
        t_invw = 1.0 / (w_v + 1e-8)
        t_ylow = h_v
        t_dy = _gather(h_v, lane1) - h_v
        s_hi = _gather(s_v, lane1)
        t_slm2 = s_v - 2.0
        t_ssum = s_v + s_hi
        t_tsm2 = t_ssum + t_ssum - 4.0
        e0 = _gather(edges, jnp.full((LANES,), 0, jnp.int32))
        e1 = _gather(edges, jnp.full((LANES,), 1, jnp.int32))
        e2 = _gather(edges, jnp.full((LANES,), 2, jnp.int32))
        e3 = _gather(edges, jnp.full((LANES,), 3, jnp.int32))
        two = jnp.full((LANES,), 2.0, jnp.float32)

        def run_chunk(in_v, out_v):
            def compute(i):
                r = i >> 5
                c = (i & 31) * LANES
                xv = in_v[r, pl.ds(c, LANES)]
                xf = jnp.minimum(jnp.maximum(xv, 0.0), 1.0)
                i0 = jnp.int32(0)
                idx = jnp.where(
                    xf > e3, jnp.int32(4),
                    jnp.where(xf > e2, jnp.int32(3),
                              jnp.where(xf > e1, jnp.int32(2),
                                        jnp.where(xf > e0, jnp.int32(1), i0))),
                )
                x_low = _gather(t_xlow, idx)
                inv_w = _gather(t_invw, idx)
                y_low = _gather(t_ylow, idx)
                dy = _gather(t_dy, idx)
                slm2 = _gather(t_slm2, idx)
                ssum = _gather(t_ssum, idx)
                tsm2 = _gather(t_tsm2, idx)
                t = (xf - x_low) * inv_w
                t2 = t * t
                num = (slm2 * t + two) * t
                den = ssum * t2 + (tsm2 * t + two)
                y = y_low + dy * (num / den)
                out_v[r, pl.ds(c, LANES)] = y

            plsc.parallel_loop(0, ROWS * 32, 1, unroll=4)(compute)

        n_pair = n_chunks // 2
        pltpu.async_copy(x_hbm.at[pl.ds(base, ROWS), :], in0, si0)
        pltpu.async_copy(x_hbm.at[pl.ds(base + ROWS, ROWS), :], in1, si1)

        def chunk_pair(k, _):
            off0 = base + k * (2 * ROWS)
            for (off, in_v, out_v, si, so) in (
                (off0, in0, out0, si0, so0),
                (off0 + ROWS, in1, out1, si1, so1),
            ):
                pltpu.make_async_copy(x_hbm.at[pl.ds(off, ROWS), :], in_v, si).wait()

                @pl.when(k > 0)
                def _():
                    pltpu.make_async_copy(
                        out_v, out_hbm.at[pl.ds(off, ROWS), :], so).wait()

                run_chunk(in_v, out_v)
                pltpu.async_copy(out_v, out_hbm.at[pl.ds(off, ROWS), :], so)

                @pl.when(k < n_pair - 1)
                def _():
                    pltpu.async_copy(
                        x_hbm.at[pl.ds(off + 2 * ROWS, ROWS), :], in_v, si)
            return 0

        lax.fori_loop(0, n_pair, chunk_pair, 0)
        last0 = base + (n_chunks - 2) * ROWS
        pltpu.make_async_copy(out0, out_hbm.at[pl.ds(last0, ROWS), :], so0).wait()
        pltpu.make_async_copy(
            out1, out_hbm.at[pl.ds(last0 + ROWS, ROWS), :], so1).wait()

    return tone


TC_BLOCK_ROWS = 512


@functools.lru_cache(maxsize=None)
def _make_tc_kernel(m, m_sc):
    n_blocks = (m - m_sc) // TC_BLOCK_ROWS
    assert n_blocks * TC_BLOCK_ROWS == m - m_sc
    blk0 = m_sc // TC_BLOCK_ROWS

    def body(w_ref, h_ref, s_ref, x_ref, o_ref, p_ref):
        @pl.when(pl.program_id(0) == 0)
        def _():
            e = w_ref[0]
            p_ref[35] = e
            for b in range(1, 5):
                e = e + w_ref[b]
                p_ref[35 + b] = e
            for b in range(5):
                p_ref[0 + b] = p_ref[34 + b] if b > 0 else 0.0
                p_ref[5 + b] = 1.0 / (w_ref[b] + 1e-8)
                p_ref[10 + b] = h_ref[b]
                p_ref[15 + b] = h_ref[b + 1] - h_ref[b]
                p_ref[20 + b] = s_ref[b] - 2.0
                ss = s_ref[b] + s_ref[b + 1]
                p_ref[25 + b] = ss
                p_ref[30 + b] = 2.0 * (ss - 2.0)

        xv = x_ref[...]
        xf = jnp.minimum(jnp.maximum(xv, 0.0), 1.0)
        m0 = xf > p_ref[35]
        m1 = xf > p_ref[36]
        m2 = xf > p_ref[37]
        m3 = xf > p_ref[38]

        def sel(k):
            return jnp.where(
                m3, p_ref[k + 4],
                jnp.where(m2, p_ref[k + 3],
                          jnp.where(m1, p_ref[k + 2],
                                    jnp.where(m0, p_ref[k + 1], p_ref[k]))))

        xl = sel(0)
        iw = sel(5)
        yl = sel(10)
        dyv = sel(15)
        a = sel(20)
        sm = sel(25)
        ts = sel(30)
        t = (xf - xl) * iw
        t2 = t * t
        num = (a * t + 2.0) * t
        den = sm * t2 + (ts * t + 2.0)
        o_ref[...] = yl + dyv * (num / den)

    return pl.pallas_call(
        body,
        grid=(n_blocks,),
        in_specs=[
            pl.BlockSpec(memory_space=pltpu.SMEM),
            pl.BlockSpec(memory_space=pltpu.SMEM),
            pl.BlockSpec(memory_space=pltpu.SMEM),
            pl.BlockSpec((TC_BLOCK_ROWS, 512), lambda i: (blk0 + i, 0)),
        ],
        out_specs=pl.BlockSpec((TC_BLOCK_ROWS, 512), lambda i: (blk0 + i, 0)),
        out_shape=jax.ShapeDtypeStruct((m, 512), jnp.float32),
        scratch_shapes=[pltpu.SMEM((40,), jnp.float32)],
    )


SC_ROWS_TOTAL = 8192


def kernel(x, widths, heights, slopes):
    orig_shape = x.shape
    x2 = x.reshape(-1, 512)
    m = x2.shape[0]
    m_sc = SC_ROWS_TOTAL if m % (2 * 32 * 32) == 0 and m > SC_ROWS_TOTAL else m
    if m_sc < m:
        sc_out = _make_sc_kernel(m_sc, m)(x2, widths, heights, slopes)
        tc_out = _make_tc_kernel(m, m_sc)(widths, heights, slopes, x2)
        out = jax.lax.dynamic_update_slice(tc_out, sc_out, (0, 0))
    else:
        out = _make_sc_kernel(m, m)(x2, widths, heights, slopes)
    return out.reshape(orig_shape)


# speedup vs baseline: 1.6618x; 1.0015x over previous
import functools

import jax
import jax.numpy as jnp
from jax import lax
from jax.experimental import pallas as pl
from jax.experimental.pallas import tpu as pltpu
from jax.experimental.pallas import tpu_sc as plsc

NUM_WORKERS = 32
LANES = 16

ROWS = 32


@functools.lru_cache(maxsize=None)
def _make_sc_kernel(m_sc, m):
    rows_w = m_sc // NUM_WORKERS
    assert rows_w * NUM_WORKERS == m_sc and rows_w % (2 * ROWS) == 0
    n_chunks = rows_w // ROWS
    mesh = plsc.VectorSubcoreMesh(core_axis_name="c", subcore_axis_name="s")

    @functools.partial(
        pl.kernel,
        out_type=jax.ShapeDtypeStruct((m_sc, 512), jnp.float32),
        mesh=mesh,
        compiler_params=pltpu.CompilerParams(use_tc_tiling_on_sc=True),
        scratch_types=[
            pltpu.VMEM((3 * LANES,), jnp.float32),
            pltpu.VMEM((ROWS, 512), jnp.float32),
            pltpu.VMEM((ROWS, 512), jnp.float32),
            pltpu.VMEM((ROWS, 512), jnp.float32),
            pltpu.VMEM((ROWS, 512), jnp.float32),
            pltpu.SemaphoreType.DMA,
            pltpu.SemaphoreType.DMA,
            pltpu.SemaphoreType.DMA,
            pltpu.SemaphoreType.DMA,
        ],
    )
    def tone(x_hbm, w_hbm, h_hbm, s_hbm, out_hbm, whs_v, in0, in1, out0, out1,
             si0, si1, so0, so1):
        wid = lax.axis_index("s") * 2 + lax.axis_index("c")
        base = wid * rows_w
        pltpu.sync_copy(w_hbm, whs_v.at[pl.ds(0, 5)])
        pltpu.sync_copy(h_hbm, whs_v.at[pl.ds(LANES, 6)])
        pltpu.sync_copy(s_hbm, whs_v.at[pl.ds(2 * LANES, 6)])

        gdims = lax.GatherDimensionNumbers(
            offset_dims=(), collapsed_slice_dims=(0,), start_index_map=(0,))

        def _gather(tab, idx):
            return lax.gather(
                tab, idx[:, None], gdims, (1,),
                mode=lax.GatherScatterMode.PROMISE_IN_BOUNDS)

        w_v = whs_v[pl.ds(0, LANES)]
        h_v = whs_v[pl.ds(LANES, LANES)]
        s_v = whs_v[pl.ds(2 * LANES, LANES)]
        lane = lax.iota(jnp.int32, LANES)
        lane1 = jnp.minimum(lane + 1, LANES - 1)
        edges = w_v
        for k in (1, 2, 4):
            shifted = _gather(edges, jnp.maximum(lane - k, 0))
            edges = edges + jnp.where(lane >= k, shifted, 0.0)
        t_xlow = jnp.where(lane == 0, 0.0,
                           _gather(edges, jnp.maximum(lane - 1, 0)))
        t_invw = 1.0 / (w_v + 1e-8)
        t_ylow = h_v
        t_dy = _gather(h_v, lane1) - h_v
        s_hi = _gather(s_v, lane1)
        t_slm2 = s_v - 2.0
        t_ssum = s_v + s_hi
        t_tsm2 = t_ssum + t_ssum - 4.0
        e0 = _gather(edges, jnp.full((LANES,), 0, jnp.int32))
        e1 = _gather(edges, jnp.full((LANES,), 1, jnp.int32))
        e2 = _gather(edges, jnp.full((LANES,), 2, jnp.int32))
        e3 = _gather(edges, jnp.full((LANES,), 3, jnp.int32))
        two = jnp.full((LANES,), 2.0, jnp.float32)

        def run_chunk(in_v, out_v):
            def compute(i):
                r = i >> 5
                c = (i & 31) * LANES
                xv = in_v[r, pl.ds(c, LANES)]
                xf = jnp.minimum(jnp.maximum(xv, 0.0), 1.0)
                i0 = jnp.int32(0)
                idx = jnp.where(
                    xf > e3, jnp.int32(4),
                    jnp.where(xf > e2, jnp.int32(3),
                              jnp.where(xf > e1, jnp.int32(2),
                                        jnp.where(xf > e0, jnp.int32(1), i0))),
                )
                x_low = _gather(t_xlow, idx)
                inv_w = _gather(t_invw, idx)
                y_low = _gather(t_ylow, idx)
                dy = _gather(t_dy, idx)
                slm2 = _gather(t_slm2, idx)
                ssum = _gather(t_ssum, idx)
                tsm2 = _gather(t_tsm2, idx)
                t = (xf - x_low) * inv_w
                t2 = t * t
                num = (slm2 * t + two) * t
                den = ssum * t2 + (tsm2 * t + two)
                y = y_low + dy * (num / den)
                out_v[r, pl.ds(c, LANES)] = y

            plsc.parallel_loop(0, ROWS * 32, 1, unroll=4)(compute)

        n_pair = n_chunks // 2
        pltpu.async_copy(x_hbm.at[pl.ds(base, ROWS), :], in0, si0)
        pltpu.async_copy(x_hbm.at[pl.ds(base + ROWS, ROWS), :], in1, si1)

        def chunk_pair(k, _):
            off0 = base + k * (2 * ROWS)
            for (off, in_v, out_v, si, so) in (
                (off0, in0, out0, si0, so0),
                (off0 + ROWS, in1, out1, si1, so1),
            ):
                pltpu.make_async_copy(x_hbm.at[pl.ds(off, ROWS), :], in_v, si).wait()

                @pl.when(k > 0)
                def _():
                    pltpu.make_async_copy(
                        out_v, out_hbm.at[pl.ds(off, ROWS), :], so).wait()

                run_chunk(in_v, out_v)
                pltpu.async_copy(out_v, out_hbm.at[pl.ds(off, ROWS), :], so)

                @pl.when(k < n_pair - 1)
                def _():
                    pltpu.async_copy(
                        x_hbm.at[pl.ds(off + 2 * ROWS, ROWS), :], in_v, si)
            return 0

        lax.fori_loop(0, n_pair, chunk_pair, 0)
        last0 = base + (n_chunks - 2) * ROWS
        pltpu.make_async_copy(out0, out_hbm.at[pl.ds(last0, ROWS), :], so0).wait()
        pltpu.make_async_copy(
            out1, out_hbm.at[pl.ds(last0 + ROWS, ROWS), :], so1).wait()

    return tone


TC_BLOCK_ROWS = 1024


@functools.lru_cache(maxsize=None)
def _make_tc_kernel(m, m_sc):
    n_blocks = (m - m_sc) // TC_BLOCK_ROWS
    assert n_blocks * TC_BLOCK_ROWS == m - m_sc
    blk0 = m_sc // TC_BLOCK_ROWS

    def body(w_ref, h_ref, s_ref, x_ref, o_ref, p_ref):
        @pl.when(pl.program_id(0) == 0)
        def _():
            e = w_ref[0]
            p_ref[35] = e
            for b in range(1, 5):
                e = e + w_ref[b]
                p_ref[35 + b] = e
            for b in range(5):
                p_ref[0 + b] = p_ref[34 + b] if b > 0 else 0.0
                p_ref[5 + b] = 1.0 / (w_ref[b] + 1e-8)
                p_ref[10 + b] = h_ref[b]
                p_ref[15 + b] = h_ref[b + 1] - h_ref[b]
                p_ref[20 + b] = s_ref[b] - 2.0
                ss = s_ref[b] + s_ref[b + 1]
                p_ref[25 + b] = ss
                p_ref[30 + b] = 2.0 * (ss - 2.0)

        xv = x_ref[...]
        xf = jnp.minimum(jnp.maximum(xv, 0.0), 1.0)
        m0 = xf > p_ref[35]
        m1 = xf > p_ref[36]
        m2 = xf > p_ref[37]
        m3 = xf > p_ref[38]

        def sel(k):
            return jnp.where(
                m3, p_ref[k + 4],
                jnp.where(m2, p_ref[k + 3],
                          jnp.where(m1, p_ref[k + 2],
                                    jnp.where(m0, p_ref[k + 1], p_ref[k]))))

        xl = sel(0)
        iw = sel(5)
        yl = sel(10)
        dyv = sel(15)
        a = sel(20)
        sm = sel(25)
        ts = sel(30)
        t = (xf - xl) * iw
        t2 = t * t
        num = (a * t + 2.0) * t
        den = sm * t2 + (ts * t + 2.0)
        o_ref[...] = yl + dyv * (num / den)

    return pl.pallas_call(
        body,
        grid=(n_blocks,),
        in_specs=[
            pl.BlockSpec(memory_space=pltpu.SMEM),
            pl.BlockSpec(memory_space=pltpu.SMEM),
            pl.BlockSpec(memory_space=pltpu.SMEM),
            pl.BlockSpec((TC_BLOCK_ROWS, 512), lambda i: (blk0 + i, 0)),
        ],
        out_specs=pl.BlockSpec((TC_BLOCK_ROWS, 512), lambda i: (blk0 + i, 0)),
        out_shape=jax.ShapeDtypeStruct((m, 512), jnp.float32),
        scratch_shapes=[pltpu.SMEM((40,), jnp.float32)],
    )


SC_ROWS_TOTAL = 8192


def kernel(x, widths, heights, slopes):
    orig_shape = x.shape
    x2 = x.reshape(-1, 512)
    m = x2.shape[0]
    m_sc = SC_ROWS_TOTAL if m % (2 * 32 * 32) == 0 and m > SC_ROWS_TOTAL else m
    if m_sc < m:
        sc_out = _make_sc_kernel(m_sc, m)(x2, widths, heights, slopes)
        tc_out = _make_tc_kernel(m, m_sc)(widths, heights, slopes, x2)
        out = jax.lax.dynamic_update_slice(tc_out, sc_out, (0, 0))
    else:
        out = _make_sc_kernel(m, m)(x2, widths, heights, slopes)
    return out.reshape(orig_shape)


# speedup vs baseline: 1.6657x; 1.0023x over previous
import functools

import jax
import jax.numpy as jnp
from jax import lax
from jax.experimental import pallas as pl
from jax.experimental.pallas import tpu as pltpu
from jax.experimental.pallas import tpu_sc as plsc

NUM_WORKERS = 32
LANES = 16

ROWS = 32


@functools.lru_cache(maxsize=None)
def _make_sc_kernel(m_sc, m):
    rows_w = m_sc // NUM_WORKERS
    assert rows_w * NUM_WORKERS == m_sc and rows_w % (2 * ROWS) == 0
    n_chunks = rows_w // ROWS
    mesh = plsc.VectorSubcoreMesh(core_axis_name="c", subcore_axis_name="s")

    @functools.partial(
        pl.kernel,
        out_type=jax.ShapeDtypeStruct((m_sc, 512), jnp.float32),
        mesh=mesh,
        compiler_params=pltpu.CompilerParams(use_tc_tiling_on_sc=True),
        scratch_types=[
            pltpu.VMEM((3 * LANES,), jnp.float32),
            pltpu.VMEM((ROWS, 512), jnp.float32),
            pltpu.VMEM((ROWS, 512), jnp.float32),
            pltpu.VMEM((ROWS, 512), jnp.float32),
            pltpu.VMEM((ROWS, 512), jnp.float32),
            pltpu.SemaphoreType.DMA,
            pltpu.SemaphoreType.DMA,
            pltpu.SemaphoreType.DMA,
            pltpu.SemaphoreType.DMA,
        ],
    )
    def tone(x_hbm, w_hbm, h_hbm, s_hbm, out_hbm, whs_v, in0, in1, out0, out1,
             si0, si1, so0, so1):
        wid = lax.axis_index("s") * 2 + lax.axis_index("c")
        base = wid * rows_w
        pltpu.sync_copy(w_hbm, whs_v.at[pl.ds(0, 5)])
        pltpu.sync_copy(h_hbm, whs_v.at[pl.ds(LANES, 6)])
        pltpu.sync_copy(s_hbm, whs_v.at[pl.ds(2 * LANES, 6)])

        gdims = lax.GatherDimensionNumbers(
            offset_dims=(), collapsed_slice_dims=(0,), start_index_map=(0,))

        def _gather(tab, idx):
            return lax.gather(
                tab, idx[:, None], gdims, (1,),
                mode=lax.GatherScatterMode.PROMISE_IN_BOUNDS)

        w_v = whs_v[pl.ds(0, LANES)]
        h_v = whs_v[pl.ds(LANES, LANES)]
        s_v = whs_v[pl.ds(2 * LANES, LANES)]
        lane = lax.iota(jnp.int32, LANES)
        lane1 = jnp.minimum(lane + 1, LANES - 1)
        edges = w_v
        for k in (1, 2, 4):
            shifted = _gather(edges, jnp.maximum(lane - k, 0))
            edges = edges + jnp.where(lane >= k, shifted, 0.0)
        t_xlow = jnp.where(lane == 0, 0.0,
                           _gather(edges, jnp.maximum(lane - 1, 0)))
        t_invw = 1.0 / (w_v + 1e-8)
        t_ylow = h_v
        t_dy = _gather(h_v, lane1) - h_v
        s_hi = _gather(s_v, lane1)
        t_slm2 = s_v - 2.0
        t_ssum = s_v + s_hi
        t_tsm2 = t_ssum + t_ssum - 4.0
        e0 = _gather(edges, jnp.full((LANES,), 0, jnp.int32))
        e1 = _gather(edges, jnp.full((LANES,), 1, jnp.int32))
        e2 = _gather(edges, jnp.full((LANES,), 2, jnp.int32))
        e3 = _gather(edges, jnp.full((LANES,), 3, jnp.int32))
        two = jnp.full((LANES,), 2.0, jnp.float32)

        def run_chunk(in_v, out_v):
            def compute(i):
                r = i >> 5
                c = (i & 31) * LANES
                xv = in_v[r, pl.ds(c, LANES)]
                xf = jnp.minimum(jnp.maximum(xv, 0.0), 1.0)
                i0 = jnp.int32(0)
                idx = jnp.where(
                    xf > e3, jnp.int32(4),
                    jnp.where(xf > e2, jnp.int32(3),
                              jnp.where(xf > e1, jnp.int32(2),
                                        jnp.where(xf > e0, jnp.int32(1), i0))),
                )
                x_low = _gather(t_xlow, idx)
                inv_w = _gather(t_invw, idx)
                y_low = _gather(t_ylow, idx)
                dy = _gather(t_dy, idx)
                slm2 = _gather(t_slm2, idx)
                ssum = _gather(t_ssum, idx)
                tsm2 = _gather(t_tsm2, idx)
                t = (xf - x_low) * inv_w
                t2 = t * t
                num = (slm2 * t + two) * t
                den = ssum * t2 + (tsm2 * t + two)
                y = y_low + dy * (num / den)
                out_v[r, pl.ds(c, LANES)] = y

            plsc.parallel_loop(0, ROWS * 32, 1, unroll=4)(compute)

        n_pair = n_chunks // 2
        pltpu.async_copy(x_hbm.at[pl.ds(base, ROWS), :], in0, si0)
        pltpu.async_copy(x_hbm.at[pl.ds(base + ROWS, ROWS), :], in1, si1)

        def chunk_pair(k, _):
            off0 = base + k * (2 * ROWS)
            for (off, in_v, out_v, si, so) in (
                (off0, in0, out0, si0, so0),
                (off0 + ROWS, in1, out1, si1, so1),
            ):
                pltpu.make_async_copy(x_hbm.at[pl.ds(off, ROWS), :], in_v, si).wait()

                @pl.when(k > 0)
                def _():
                    pltpu.make_async_copy(
                        out_v, out_hbm.at[pl.ds(off, ROWS), :], so).wait()

                run_chunk(in_v, out_v)
                pltpu.async_copy(out_v, out_hbm.at[pl.ds(off, ROWS), :], so)

                @pl.when(k < n_pair - 1)
                def _():
                    pltpu.async_copy(
                        x_hbm.at[pl.ds(off + 2 * ROWS, ROWS), :], in_v, si)
            return 0

        lax.fori_loop(0, n_pair, chunk_pair, 0)
        last0 = base + (n_chunks - 2) * ROWS
        pltpu.make_async_copy(out0, out_hbm.at[pl.ds(last0, ROWS), :], so0).wait()
        pltpu.make_async_copy(
            out1, out_hbm.at[pl.ds(last0 + ROWS, ROWS), :], so1).wait()

    return tone


TC_BLOCK_ROWS = 1024


@functools.lru_cache(maxsize=None)
def _make_tc_kernel(m, m_sc):
    n_blocks = (m - m_sc) // TC_BLOCK_ROWS
    assert n_blocks * TC_BLOCK_ROWS == m - m_sc
    blk0 = m_sc // TC_BLOCK_ROWS

    def body(w_ref, h_ref, s_ref, x_ref, o_ref, p_ref):
        @pl.when(pl.program_id(0) == 0)
        def _():
            e = w_ref[0]
            p_ref[35] = e
            for b in range(1, 5):
                e = e + w_ref[b]
                p_ref[35 + b] = e
            for b in range(5):
                p_ref[0 + b] = p_ref[34 + b] if b > 0 else 0.0
                p_ref[5 + b] = 1.0 / (w_ref[b] + 1e-8)
                p_ref[10 + b] = h_ref[b]
                p_ref[15 + b] = h_ref[b + 1] - h_ref[b]
                p_ref[20 + b] = s_ref[b] - 2.0
                ss = s_ref[b] + s_ref[b + 1]
                p_ref[25 + b] = ss
                p_ref[30 + b] = 2.0 * (ss - 2.0)

        xv = x_ref[...]
        xf = jnp.minimum(jnp.maximum(xv, 0.0), 1.0)
        m0 = xf > p_ref[35]
        m1 = xf > p_ref[36]
        m2 = xf > p_ref[37]
        m3 = xf > p_ref[38]

        def sel(k):
            return jnp.where(
                m3, p_ref[k + 4],
                jnp.where(m2, p_ref[k + 3],
                          jnp.where(m1, p_ref[k + 2],
                                    jnp.where(m0, p_ref[k + 1], p_ref[k]))))

        xl = sel(0)
        iw = sel(5)
        yl = sel(10)
        dyv = sel(15)
        a = sel(20)
        sm = sel(25)
        ts = sm + sm - 4.0
        t = (xf - xl) * iw
        t2 = t * t
        num = (a * t + 2.0) * t
        den = sm * t2 + (ts * t + 2.0)
        o_ref[...] = yl + dyv * (num / den)

    return pl.pallas_call(
        body,
        grid=(n_blocks,),
        in_specs=[
            pl.BlockSpec(memory_space=pltpu.SMEM),
            pl.BlockSpec(memory_space=pltpu.SMEM),
            pl.BlockSpec(memory_space=pltpu.SMEM),
            pl.BlockSpec((TC_BLOCK_ROWS, 512), lambda i: (blk0 + i, 0)),
        ],
        out_specs=pl.BlockSpec((TC_BLOCK_ROWS, 512), lambda i: (blk0 + i, 0)),
        out_shape=jax.ShapeDtypeStruct((m, 512), jnp.float32),
        scratch_shapes=[pltpu.SMEM((40,), jnp.float32)],
    )


SC_ROWS_TOTAL = 8192


def kernel(x, widths, heights, slopes):
    orig_shape = x.shape
    x2 = x.reshape(-1, 512)
    m = x2.shape[0]
    m_sc = SC_ROWS_TOTAL if m % (2 * 32 * 32) == 0 and m > SC_ROWS_TOTAL else m
    if m_sc < m:
        sc_out = _make_sc_kernel(m_sc, m)(x2, widths, heights, slopes)
        tc_out = _make_tc_kernel(m, m_sc)(widths, heights, slopes, x2)
        out = jax.lax.dynamic_update_slice(tc_out, sc_out, (0, 0))
    else:
        out = _make_sc_kernel(m, m)(x2, widths, heights, slopes)
    return out.reshape(orig_shape)


# speedup vs baseline: 1.6663x; 1.0004x over previous
import functools

import jax
import jax.numpy as jnp
from jax import lax
from jax.experimental import pallas as pl
from jax.experimental.pallas import tpu as pltpu
from jax.experimental.pallas import tpu_sc as plsc

NUM_WORKERS = 32
LANES = 16

ROWS = 32


@functools.lru_cache(maxsize=None)
def _make_sc_kernel(m_sc, m):
    rows_w = m_sc // NUM_WORKERS
    assert rows_w * NUM_WORKERS == m_sc and rows_w % (2 * ROWS) == 0
    n_chunks = rows_w // ROWS
    mesh = plsc.VectorSubcoreMesh(core_axis_name="c", subcore_axis_name="s")

    @functools.partial(
        pl.kernel,
        out_type=jax.ShapeDtypeStruct((m_sc, 512), jnp.float32),
        mesh=mesh,
        compiler_params=pltpu.CompilerParams(use_tc_tiling_on_sc=True),
        scratch_types=[
            pltpu.VMEM((3 * LANES,), jnp.float32),
            pltpu.VMEM((ROWS, 512), jnp.float32),
            pltpu.VMEM((ROWS, 512), jnp.float32),
            pltpu.VMEM((ROWS, 512), jnp.float32),
            pltpu.VMEM((ROWS, 512), jnp.float32),
            pltpu.SemaphoreType.DMA,
            pltpu.SemaphoreType.DMA,
            pltpu.SemaphoreType.DMA,
            pltpu.SemaphoreType.DMA,
        ],
    )
    def tone(x_hbm, w_hbm, h_hbm, s_hbm, out_hbm, whs_v, in0, in1, out0, out1,
             si0, si1, so0, so1):
        wid = lax.axis_index("s") * 2 + lax.axis_index("c")
        base = wid * rows_w
        pltpu.sync_copy(w_hbm, whs_v.at[pl.ds(0, 5)])
        pltpu.sync_copy(h_hbm, whs_v.at[pl.ds(LANES, 6)])
        pltpu.sync_copy(s_hbm, whs_v.at[pl.ds(2 * LANES, 6)])

        gdims = lax.GatherDimensionNumbers(
            offset_dims=(), collapsed_slice_dims=(0,), start_index_map=(0,))

        def _gather(tab, idx):
            return lax.gather(
                tab, idx[:, None], gdims, (1,),
                mode=lax.GatherScatterMode.PROMISE_IN_BOUNDS)

        w_v = whs_v[pl.ds(0, LANES)]
        h_v = whs_v[pl.ds(LANES, LANES)]
        s_v = whs_v[pl.ds(2 * LANES, LANES)]
        lane = lax.iota(jnp.int32, LANES)
        lane1 = jnp.minimum(lane + 1, LANES - 1)
        edges = w_v
        for k in (1, 2, 4):
            shifted = _gather(edges, jnp.maximum(lane - k, 0))
            edges = edges + jnp.where(lane >= k, shifted, 0.0)
        t_xlow = jnp.where(lane == 0, 0.0,
                           _gather(edges, jnp.maximum(lane - 1, 0)))
        t_invw = 1.0 / (w_v + 1e-8)
        t_ylow = h_v
        t_dy = _gather(h_v, lane1) - h_v
        s_hi = _gather(s_v, lane1)
        t_slm2 = s_v - 2.0
        t_ssum = s_v + s_hi
        t_tsm2 = t_ssum + t_ssum - 4.0
        e0 = _gather(edges, jnp.full((LANES,), 0, jnp.int32))
        e1 = _gather(edges, jnp.full((LANES,), 1, jnp.int32))
        e2 = _gather(edges, jnp.full((LANES,), 2, jnp.int32))
        e3 = _gather(edges, jnp.full((LANES,), 3, jnp.int32))
        two = jnp.full((LANES,), 2.0, jnp.float32)

        def run_chunk(in_v, out_v):
            def compute(i):
                r = i >> 5
                c = (i & 31) * LANES
                xv = in_v[r, pl.ds(c, LANES)]
                xf = jnp.minimum(jnp.maximum(xv, 0.0), 1.0)
                i0 = jnp.int32(0)
                idx = jnp.where(
                    xf > e3, jnp.int32(4),
                    jnp.where(xf > e2, jnp.int32(3),
                              jnp.where(xf > e1, jnp.int32(2),
                                        jnp.where(xf > e0, jnp.int32(1), i0))),
                )
                x_low = _gather(t_xlow, idx)
                inv_w = _gather(t_invw, idx)
                y_low = _gather(t_ylow, idx)
                dy = _gather(t_dy, idx)
                slm2 = _gather(t_slm2, idx)
                ssum = _gather(t_ssum, idx)
                tsm2 = _gather(t_tsm2, idx)
                t = (xf - x_low) * inv_w
                t2 = t * t
                num = (slm2 * t + two) * t
                den = ssum * t2 + (tsm2 * t + two)
                y = y_low + dy * (num / den)
                out_v[r, pl.ds(c, LANES)] = y

            plsc.parallel_loop(0, ROWS * 32, 1, unroll=4)(compute)

        n_pair = n_chunks // 2
        pltpu.async_copy(x_hbm.at[pl.ds(base, ROWS), :], in0, si0)
        pltpu.async_copy(x_hbm.at[pl.ds(base + ROWS, ROWS), :], in1, si1)

        def chunk_pair(k, _):
            off0 = base + k * (2 * ROWS)
            for (off, in_v, out_v, si, so) in (
                (off0, in0, out0, si0, so0),
                (off0 + ROWS, in1, out1, si1, so1),
            ):
                pltpu.make_async_copy(x_hbm.at[pl.ds(off, ROWS), :], in_v, si).wait()

                @pl.when(k > 0)
                def _():
                    pltpu.make_async_copy(
                        out_v, out_hbm.at[pl.ds(off, ROWS), :], so).wait()

                run_chunk(in_v, out_v)
                pltpu.async_copy(out_v, out_hbm.at[pl.ds(off, ROWS), :], so)

                @pl.when(k < n_pair - 1)
                def _():
                    pltpu.async_copy(
                        x_hbm.at[pl.ds(off + 2 * ROWS, ROWS), :], in_v, si)
            return 0

        lax.fori_loop(0, n_pair, chunk_pair, 0)
        last0 = base + (n_chunks - 2) * ROWS
        pltpu.make_async_copy(out0, out_hbm.at[pl.ds(last0, ROWS), :], so0).wait()
        pltpu.make_async_copy(
            out1, out_hbm.at[pl.ds(last0 + ROWS, ROWS), :], so1).wait()

    return tone


TC_BLOCK_ROWS = 1024


@functools.lru_cache(maxsize=None)
def _make_tc_kernel(m, m_sc):
    n_blocks = (m - m_sc) // TC_BLOCK_ROWS
    assert n_blocks * TC_BLOCK_ROWS == m - m_sc
    blk0 = m_sc // TC_BLOCK_ROWS

    def body(w_ref, h_ref, s_ref, x_ref, o_ref, p_ref):
        @pl.when(pl.program_id(0) == 0)
        def _():
            e = w_ref[0]
            p_ref[35] = e
            for b in range(1, 5):
                e = e + w_ref[b]
                p_ref[35 + b] = e
            for b in range(5):
                p_ref[0 + b] = p_ref[34 + b] if b > 0 else 0.0
                p_ref[5 + b] = 1.0 / (w_ref[b] + 1e-8)
                p_ref[10 + b] = h_ref[b]
                p_ref[15 + b] = h_ref[b + 1] - h_ref[b]
                p_ref[20 + b] = s_ref[b] - 2.0
                p_ref[25 + b] = s_ref[b] + s_ref[b + 1]

        xv = x_ref[...]
        xf = jnp.minimum(jnp.maximum(xv, 0.0), 1.0)
        m0 = xf > p_ref[35]
        m1 = xf > p_ref[36]
        m2 = xf > p_ref[37]
        m3 = xf > p_ref[38]

        def sel(k):
            return jnp.where(
                m3, p_ref[k + 4],
                jnp.where(m2, p_ref[k + 3],
                          jnp.where(m1, p_ref[k + 2],
                                    jnp.where(m0, p_ref[k + 1], p_ref[k]))))

        xl = sel(0)
        iw = sel(5)
        yl = sel(10)
        dyv = sel(15)
        a = sel(20)
        sm = sel(25)
        ts = sm + sm - 4.0
        t = (xf - xl) * iw
        t2 = t * t
        num = (a * t + 2.0) * t
        den = sm * t2 + (ts * t + 2.0)
        o_ref[...] = yl + dyv * (num / den)

    return pl.pallas_call(
        body,
        grid=(n_blocks,),
        in_specs=[
            pl.BlockSpec(memory_space=pltpu.SMEM),
            pl.BlockSpec(memory_space=pltpu.SMEM),
            pl.BlockSpec(memory_space=pltpu.SMEM),
            pl.BlockSpec((TC_BLOCK_ROWS, 512), lambda i: (blk0 + i, 0)),
        ],
        out_specs=pl.BlockSpec((TC_BLOCK_ROWS, 512), lambda i: (blk0 + i, 0)),
        out_shape=jax.ShapeDtypeStruct((m, 512), jnp.float32),
        scratch_shapes=[pltpu.SMEM((40,), jnp.float32)],
    )


SC_ROWS_TOTAL = 8192


def kernel(x, widths, heights, slopes):
    orig_shape = x.shape
    x2 = x.reshape(-1, 512)
    m = x2.shape[0]
    m_sc = SC_ROWS_TOTAL if m % (2 * 32 * 32) == 0 and m > SC_ROWS_TOTAL else m
    if m_sc < m:
        sc_out = _make_sc_kernel(m_sc, m)(x2, widths, heights, slopes)
        tc_out = _make_tc_kernel(m, m_sc)(widths, heights, slopes, x2)
        out = jax.lax.dynamic_update_slice(tc_out, sc_out, (0, 0))
    else:
        out = _make_sc_kernel(m, m)(x2, widths, heights, slopes)
    return out.reshape(orig_shape)
